# baseline (speedup 1.0000x reference)
import functools

import jax
import jax.numpy as jnp
import numpy as np
from jax import lax
from jax.experimental import pallas as pl
from jax.experimental.pallas import tpu as pltpu
from jax.experimental.pallas import tpu_sc as plsc

N = 10000
E = 320000
D = 128
H = 128
S = 8
O = 64

NC = 2
NS = 16
NP = 10112
RPT = NP // NS
CHUNK = 128
EPT = E // NS
NCH = 160
NGCH = NCH // 2
NGPAIR = NGCH // 2
EPT_PAD = NCH * CHUNK
RB = 400


def _hippo_constants():
    order, dt = O, 1.0 / 512.0
    Q = np.arange(order, dtype=np.float64)
    R = np.sqrt(2.0 * Q + 1.0)
    i, j = np.meshgrid(Q, Q, indexing='ij')
    A = np.where(i < j, (-1.0) ** (i - j), 1.0) * R[:, None] * R[None, :]
    A = -A
    B = R[:, None]
    I = np.eye(order)
    Ad = np.linalg.solve(I - dt / 2.0 * A, I + dt / 2.0 * A)
    Bd = np.linalg.solve(I - dt / 2.0 * A, dt * B)[:, 0]
    return jnp.asarray(Ad, jnp.float32), jnp.asarray(Bd, jnp.float32)


def _a_body(x_r, h_r, m_r, px_r, ph_r, pm_r, c_r, y1_r, sg_r, sc_r):
    acc = jnp.dot(x_r[...], px_r[...], preferred_element_type=jnp.float32)
    acc = acc + jnp.dot(h_r[...], ph_r[...], preferred_element_type=jnp.float32)
    acc = acc + jnp.dot(m_r[...], pm_r[...], preferred_element_type=jnp.float32)
    acc = acc + c_r[...]
    for k in range(6):
        y1_r[k] = acc[:, 64 * k:64 * (k + 1)]
    sg_r[...] = acc[:, 384:640]
    sc_r[...] = acc[:, 640:768]


def _mm_a(x, h, m2, Px, Ph, Pm, cv):
    K = 768
    return pl.pallas_call(
        _a_body,
        grid=(N // RB,),
        in_specs=[
            pl.BlockSpec((RB, D), lambda i: (i, 0)),
            pl.BlockSpec((RB, H), lambda i: (i, 0)),
            pl.BlockSpec((RB, S * O), lambda i: (i, 0)),
            pl.BlockSpec((D, K), lambda i: (0, 0)),
            pl.BlockSpec((H, K), lambda i: (0, 0)),
            pl.BlockSpec((S * O, K), lambda i: (0, 0)),
            pl.BlockSpec((1, K), lambda i: (0, 0)),
        ],
        out_specs=(pl.BlockSpec((6, RB, 64), lambda i: (0, i, 0)),
                   pl.BlockSpec((RB, 2 * H), lambda i: (i, 0)),
                   pl.BlockSpec((RB, H), lambda i: (i, 0))),
        out_shape=(jax.ShapeDtypeStruct((6, N, 64), jnp.float32),
                   jax.ShapeDtypeStruct((N, 2 * H), jnp.float32),
                   jax.ShapeDtypeStruct((N, H), jnp.float32)),
    )(x, h, m2, Px, Ph, Pm, cv.reshape(1, K))


def _b_body(sg_r, agg1_r, deg_r, h_r, scxm_r, w2_r, y2_r, prec_r, z_r):
    invd = 1.0 / jnp.maximum(deg_r[...], 1.0)
    acat = jnp.concatenate([agg1_r[k] for k in range(6)], axis=1)
    g = jax.nn.sigmoid(sg_r[...] + acat[:, :2 * H] * invd)
    z = g[:, :H]
    r = g[:, H:]
    rh = r * h_r[...]
    y2w = jnp.dot(rh, w2_r[...], preferred_element_type=jnp.float32)
    y2_r[0] = y2w[:, :64]
    y2_r[1] = y2w[:, 64:H]
    prec_r[...] = scxm_r[...] + y2w[:, H:] + acat[:, 2 * H:] * invd
    z_r[...] = z


def _mm_b(sg, agg1, deg, h, scxm, W2):
    outs = (jax.ShapeDtypeStruct((2, N, 64), jnp.float32),
            jax.ShapeDtypeStruct((N, H), jnp.float32),
            jax.ShapeDtypeStruct((N, H), jnp.float32))
    return pl.pallas_call(
        _b_body,
        grid=(N // RB,),
        in_specs=[
            pl.BlockSpec((RB, 2 * H), lambda i: (i, 0)),
            pl.BlockSpec((6, RB, 64), lambda i: (0, i, 0)),
            pl.BlockSpec((RB, 1), lambda i: (i, 0)),
            pl.BlockSpec((RB, H), lambda i: (i, 0)),
            pl.BlockSpec((RB, H), lambda i: (i, 0)),
            pl.BlockSpec((H, 2 * H), lambda i: (0, 0)),
        ],
        out_specs=(pl.BlockSpec((2, RB, 64), lambda i: (0, i, 0)),
                   pl.BlockSpec((RB, H), lambda i: (i, 0)),
                   pl.BlockSpec((RB, H), lambda i: (i, 0))),
        out_shape=outs,
    )(sg, agg1, deg, h, scxm, W2)


def _c_body(z_r, prec_r, agg2_r, deg_r, h_r, o_r):
    invd = 1.0 / jnp.maximum(deg_r[...], 1.0)
    acat = jnp.concatenate([agg2_r[0], agg2_r[1]], axis=1)
    c = jnp.tanh(prec_r[...] + acat * invd)
    z = z_r[...]
    o_r[...] = (1.0 - z) * h_r[...] + z * c


def _mm_c(z, prec, agg2, deg, h):
    return pl.pallas_call(
        _c_body,
        grid=(N // RB,),
        in_specs=[
            pl.BlockSpec((RB, H), lambda i: (i, 0)),
            pl.BlockSpec((RB, H), lambda i: (i, 0)),
            pl.BlockSpec((2, RB, 64), lambda i: (0, i, 0)),
            pl.BlockSpec((RB, 1), lambda i: (i, 0)),
            pl.BlockSpec((RB, H), lambda i: (i, 0)),
        ],
        out_specs=pl.BlockSpec((RB, H), lambda i: (i, 0)),
        out_shape=jax.ShapeDtypeStruct((N, H), jnp.float32),
    )(z, prec, agg2, deg, h)


@functools.cache
def _make_sc_segsum(width, ngroups, with_deg):
    mesh = plsc.VectorSubcoreMesh(core_axis_name="c", subcore_axis_name="s",
                                  num_cores=NC, num_subcores=NS)
    out_type = [jax.ShapeDtypeStruct((NC * ngroups, NP, width), jnp.float32)]
    if with_deg:
        out_type.append(jax.ShapeDtypeStruct((NP,), jnp.float32))
    scratch = [
        pltpu.VMEM((NGCH, 2 * CHUNK), jnp.int32),
        pltpu.VMEM((NCH, CHUNK), jnp.int32),
        pltpu.VMEM((2 * CHUNK, width), jnp.float32),
        pltpu.VMEM((2 * CHUNK, width), jnp.float32),
        pltpu.VMEM_SHARED((NP, width), jnp.float32),
        pltpu.SemaphoreType.DMA,
        pltpu.SemaphoreType.DMA,
    ]
    if with_deg:
        scratch += [
            pltpu.VMEM((CHUNK,), jnp.float32),
            pltpu.VMEM_SHARED((NP,), jnp.float32),
        ]

    def body(*refs):
        if with_deg:
            (tab, srci, dsti, z2, z1, ones_h, agg_o, deg_o,
             src_v, dst_v, rows0, rows1, acc, sem0, sem1,
             ones_v, dacc) = refs
        else:
            (tab, srci, dsti, z2, agg_o,
             src_v, dst_v, rows0, rows1, acc, sem0, sem1) = refs
        c = lax.axis_index("c")
        s = lax.axis_index("s")
        base = s * RPT

        pltpu.sync_copy(srci.at[s], src_v)
        pltpu.sync_copy(dsti.at[s], dst_v)
        if with_deg:
            @pl.when(c == 0)
            def _():
                pltpu.sync_copy(ones_h, ones_v)

            @pl.when((c == 0) & (s == 0))
            def _():
                pltpu.sync_copy(z1, dacc)

        for g in range(ngroups):
            q = c * ngroups + g
            pltpu.async_copy(tab.at[q].at[src_v.at[0]], rows0, sem0)
            pltpu.sync_copy(z2.at[pl.ds(base, RPT)], acc.at[pl.ds(base, RPT)])
            plsc.subcore_barrier()

            deg_now = with_deg and g == 0

            def scat(rows, j):
                for hh in range(2):
                    pltpu.sync_copy(rows.at[pl.ds(hh * CHUNK, CHUNK)],
                                    acc.at[dst_v.at[j + hh]], add=True)
                    if deg_now:
                        @pl.when(c == 0)
                        def _():
                            pltpu.sync_copy(ones_v, dacc.at[dst_v.at[j + hh]],
                                            add=True)

            def pair(t, carry):
                t0 = 2 * t
                pltpu.make_async_copy(tab.at[q].at[src_v.at[t0]], rows0,
                                      sem0).wait()
                pltpu.async_copy(tab.at[q].at[src_v.at[t0 + 1]], rows1, sem1)
                scat(rows0, 2 * t0)
                pltpu.make_async_copy(tab.at[q].at[src_v.at[t0 + 1]], rows1,
                                      sem1).wait()

                @pl.when(t + 1 < NGPAIR)
                def _():
                    pltpu.async_copy(tab.at[q].at[src_v.at[t0 + 2]], rows0,
                                     sem0)
                scat(rows1, 2 * t0 + 2)
                return carry

            lax.fori_loop(0, NGPAIR, pair, 0)
            plsc.subcore_barrier()

            pltpu.sync_copy(acc.at[pl.ds(base, RPT)],
                            agg_o.at[q].at[pl.ds(base, RPT)])
            if deg_now:
                @pl.when((c == 0) & (s == 0))
                def _():
                    pltpu.sync_copy(dacc, deg_o)

    return functools.partial(
        pl.kernel, mesh=mesh,
        out_type=tuple(out_type) if len(out_type) > 1 else out_type[0],
        compiler_params=pltpu.CompilerParams(use_tc_tiling_on_sc=False),
        scratch_types=scratch)(body)


def _fold(W, Ad, Bd):
    K = W.shape[1]
    Wb = W.reshape(S, O, K)
    Wm = jnp.einsum('po,spk->sok', Ad, Wb).reshape(S * O, K)
    Wu = jnp.einsum('p,spk->sk', Bd, Wb)
    return Wm, Wu


def kernel(x, h, m, edge_index, W_uxh, b_uxh, Wg_self, Wg_neigh, bg,
           Wc_self, Wc_neigh, bc):
    Ad, Bd = _hippo_constants()
    Wux, Wuh = W_uxh[:D], W_uxh[D:]

    def gate_cols(Wg, bias):
        Wm, Wu = _fold(Wg[D:], Ad, Bd)
        return (Wg[:D] + Wux @ Wu, Wuh @ Wu, Wm, b_uxh @ Wu + bias)

    def cand_xm_cols(Wc, bias):
        Wm, Wu = _fold(Wc[D + H:], Ad, Bd)
        return (Wc[:D] + Wux @ Wu, Wuh @ Wu, Wm, b_uxh @ Wu + bias)

    cols = [gate_cols(Wg_neigh, jnp.zeros((2 * H,), jnp.float32)),
            cand_xm_cols(Wc_neigh, jnp.zeros((H,), jnp.float32)),
            gate_cols(Wg_self, bg),
            cand_xm_cols(Wc_self, bc)]
    Px = jnp.concatenate([cc[0] for cc in cols], axis=1)
    Ph = jnp.concatenate([cc[1] for cc in cols], axis=1)
    Pm = jnp.concatenate([cc[2] for cc in cols], axis=1)
    cv = jnp.concatenate([cc[3] for cc in cols])
    W2 = jnp.concatenate([Wc_neigh[D:D + H], Wc_self[D:D + H]], axis=1)

    m2 = m.reshape(N, S * O)
    y1s, sg, scxm = _mm_a(x, h, m2, Px, Ph, Pm, cv)

    src = edge_index[0].reshape(NS, EPT)
    dst = edge_index[1].reshape(NS, EPT)
    srcT = jnp.pad(src, ((0, 0), (0, EPT_PAD - EPT))).reshape(NS, NGCH,
                                                               2 * CHUNK)
    dstT = jnp.pad(dst, ((0, 0), (0, EPT_PAD - EPT)),
                   constant_values=N).reshape(NS, NCH, CHUNK)

    zeros2 = jnp.zeros((NP, 64), jnp.float32)
    zeros1 = jnp.zeros((NP,), jnp.float32)
    ones_c = jnp.ones((CHUNK,), jnp.float32)
    agg1, deg = _make_sc_segsum(64, 3, True)(y1s, srcT, dstT, zeros2, zeros1,
                                             ones_c)
    degc = deg[:N, None]

    y2s, prec, z = _mm_b(sg, agg1, degc, h, scxm, W2)

    zeros2b = jnp.zeros((NP, 64), jnp.float32)
    agg2 = _make_sc_segsum(64, 1, False)(y2s, srcT, dstT, zeros2b)

    return _mm_c(z, prec, agg2, degc, h)


# speedup vs baseline: 1.2552x; 1.2552x over previous
import functools

import jax
import jax.numpy as jnp
import numpy as np
from jax import lax
from jax.experimental import pallas as pl
from jax.experimental.pallas import tpu as pltpu
from jax.experimental.pallas import tpu_sc as plsc

N = 10000
E = 320000
D = 128
H = 128
S = 8
O = 64

NC = 2
NS = 16
NP = 10112
RPT = NP // NS
CHUNK = 128
EPT = E // NS
NCH = 158
NPAIR = NCH // 2
EPT_PAD = NCH * CHUNK
RB = 400


def _hippo_constants():
    order, dt = O, 1.0 / 512.0
    Q = np.arange(order, dtype=np.float64)
    R = np.sqrt(2.0 * Q + 1.0)
    i, j = np.meshgrid(Q, Q, indexing='ij')
    A = np.where(i < j, (-1.0) ** (i - j), 1.0) * R[:, None] * R[None, :]
    A = -A
    B = R[:, None]
    I = np.eye(order)
    Ad = np.linalg.solve(I - dt / 2.0 * A, I + dt / 2.0 * A)
    Bd = np.linalg.solve(I - dt / 2.0 * A, dt * B)[:, 0]
    return jnp.asarray(Ad, jnp.float32), jnp.asarray(Bd, jnp.float32)


def _a_body(x_r, h_r, m_r, px_r, ph_r, pm_r, c_r, y1_r, sg_r, sc_r):
    acc = jnp.dot(x_r[...], px_r[...], preferred_element_type=jnp.float32)
    acc = acc + jnp.dot(h_r[...], ph_r[...], preferred_element_type=jnp.float32)
    acc = acc + jnp.dot(m_r[...], pm_r[...], preferred_element_type=jnp.float32)
    acc = acc + c_r[...]
    for k in range(4):
        y1_r[k] = acc[:, 96 * k:96 * (k + 1)]
    sg_r[...] = acc[:, 384:640]
    sc_r[...] = acc[:, 640:768]


def _mm_a(x, h, m2, Px, Ph, Pm, cv):
    K = 768
    return pl.pallas_call(
        _a_body,
        grid=(N // RB,),
        in_specs=[
            pl.BlockSpec((RB, D), lambda i: (i, 0)),
            pl.BlockSpec((RB, H), lambda i: (i, 0)),
            pl.BlockSpec((RB, S * O), lambda i: (i, 0)),
            pl.BlockSpec((D, K), lambda i: (0, 0)),
            pl.BlockSpec((H, K), lambda i: (0, 0)),
            pl.BlockSpec((S * O, K), lambda i: (0, 0)),
            pl.BlockSpec((1, K), lambda i: (0, 0)),
        ],
        out_specs=(pl.BlockSpec((4, RB, 96), lambda i: (0, i, 0)),
                   pl.BlockSpec((RB, 2 * H), lambda i: (i, 0)),
                   pl.BlockSpec((RB, H), lambda i: (i, 0))),
        out_shape=(jax.ShapeDtypeStruct((4, N, 96), jnp.float32),
                   jax.ShapeDtypeStruct((N, 2 * H), jnp.float32),
                   jax.ShapeDtypeStruct((N, H), jnp.float32)),
    )(x, h, m2, Px, Ph, Pm, cv.reshape(1, K))


def _b_body(sg_r, agg1_r, deg_r, h_r, scxm_r, w2_r, y2_r, prec_r, z_r):
    invd = 1.0 / jnp.maximum(deg_r[...], 1.0)
    acat = jnp.concatenate([agg1_r[k] for k in range(4)], axis=1)
    g = jax.nn.sigmoid(sg_r[...] + acat[:, :2 * H] * invd)
    z = g[:, :H]
    r = g[:, H:]
    rh = r * h_r[...]
    y2w = jnp.dot(rh, w2_r[...], preferred_element_type=jnp.float32)
    y2_r[0] = y2w[:, :64]
    y2_r[1] = y2w[:, 64:H]
    prec_r[...] = scxm_r[...] + y2w[:, H:] + acat[:, 2 * H:] * invd
    z_r[...] = z


def _mm_b(sg, agg1, deg, h, scxm, W2):
    outs = (jax.ShapeDtypeStruct((2, N, 64), jnp.float32),
            jax.ShapeDtypeStruct((N, H), jnp.float32),
            jax.ShapeDtypeStruct((N, H), jnp.float32))
    return pl.pallas_call(
        _b_body,
        grid=(N // RB,),
        in_specs=[
            pl.BlockSpec((RB, 2 * H), lambda i: (i, 0)),
            pl.BlockSpec((4, RB, 96), lambda i: (0, i, 0)),
            pl.BlockSpec((RB, 1), lambda i: (i, 0)),
            pl.BlockSpec((RB, H), lambda i: (i, 0)),
            pl.BlockSpec((RB, H), lambda i: (i, 0)),
            pl.BlockSpec((H, 2 * H), lambda i: (0, 0)),
        ],
        out_specs=(pl.BlockSpec((2, RB, 64), lambda i: (0, i, 0)),
                   pl.BlockSpec((RB, H), lambda i: (i, 0)),
                   pl.BlockSpec((RB, H), lambda i: (i, 0))),
        out_shape=outs,
    )(sg, agg1, deg, h, scxm, W2)


def _c_body(z_r, prec_r, agg2_r, deg_r, h_r, o_r):
    invd = 1.0 / jnp.maximum(deg_r[...], 1.0)
    acat = jnp.concatenate([agg2_r[0], agg2_r[1]], axis=1)
    c = jnp.tanh(prec_r[...] + acat * invd)
    z = z_r[...]
    o_r[...] = (1.0 - z) * h_r[...] + z * c


def _mm_c(z, prec, agg2, deg, h):
    return pl.pallas_call(
        _c_body,
        grid=(N // RB,),
        in_specs=[
            pl.BlockSpec((RB, H), lambda i: (i, 0)),
            pl.BlockSpec((RB, H), lambda i: (i, 0)),
            pl.BlockSpec((2, RB, 64), lambda i: (0, i, 0)),
            pl.BlockSpec((RB, 1), lambda i: (i, 0)),
            pl.BlockSpec((RB, H), lambda i: (i, 0)),
        ],
        out_specs=pl.BlockSpec((RB, H), lambda i: (i, 0)),
        out_shape=jax.ShapeDtypeStruct((N, H), jnp.float32),
    )(z, prec, agg2, deg, h)


@functools.cache
def _make_sc_segsum(width, ngroups, with_deg):
    mesh = plsc.VectorSubcoreMesh(core_axis_name="c", subcore_axis_name="s",
                                  num_cores=NC, num_subcores=NS)
    out_type = [jax.ShapeDtypeStruct((NC * ngroups, NP, width), jnp.float32)]
    if with_deg:
        out_type.append(jax.ShapeDtypeStruct((NP,), jnp.float32))
    scratch = [
        pltpu.VMEM((NCH, CHUNK), jnp.int32),
        pltpu.VMEM((NCH, CHUNK), jnp.int32),
        pltpu.VMEM((CHUNK, width), jnp.float32),
        pltpu.VMEM((CHUNK, width), jnp.float32),
        pltpu.VMEM_SHARED((NP, width), jnp.float32),
        pltpu.SemaphoreType.DMA,
        pltpu.SemaphoreType.DMA,
    ]
    if with_deg:
        scratch += [
            pltpu.VMEM((CHUNK,), jnp.float32),
            pltpu.VMEM_SHARED((NP,), jnp.float32),
        ]

    def body(*refs):
        if with_deg:
            (tab, srci, dsti, z2, z1, ones_h, agg_o, deg_o,
             src_v, dst_v, rows0, rows1, acc, sem0, sem1,
             ones_v, dacc) = refs
        else:
            (tab, srci, dsti, z2, agg_o,
             src_v, dst_v, rows0, rows1, acc, sem0, sem1) = refs
        c = lax.axis_index("c")
        s = lax.axis_index("s")
        base = s * RPT

        pltpu.sync_copy(srci.at[s], src_v)
        pltpu.sync_copy(dsti.at[s], dst_v)
        if with_deg:
            @pl.when(c == 0)
            def _():
                pltpu.sync_copy(ones_h, ones_v)

            @pl.when((c == 0) & (s == 0))
            def _():
                pltpu.sync_copy(z1, dacc)

        for g in range(ngroups):
            q = c * ngroups + g
            pltpu.async_copy(tab.at[q].at[src_v.at[0]], rows0, sem0)
            pltpu.sync_copy(z2.at[pl.ds(base, RPT)], acc.at[pl.ds(base, RPT)])
            plsc.subcore_barrier()

            deg_now = with_deg and g == 0

            def pair(t, carry):
                j0 = 2 * t
                pltpu.make_async_copy(tab.at[q].at[src_v.at[j0]], rows0,
                                      sem0).wait()
                pltpu.async_copy(tab.at[q].at[src_v.at[j0 + 1]], rows1, sem1)
                pltpu.sync_copy(rows0, acc.at[dst_v.at[j0]], add=True)
                if deg_now:
                    @pl.when(c == 0)
                    def _():
                        pltpu.sync_copy(ones_v, dacc.at[dst_v.at[j0]],
                                        add=True)
                pltpu.make_async_copy(tab.at[q].at[src_v.at[j0 + 1]], rows1,
                                      sem1).wait()

                @pl.when(t + 1 < NPAIR)
                def _():
                    pltpu.async_copy(tab.at[q].at[src_v.at[j0 + 2]], rows0,
                                     sem0)
                pltpu.sync_copy(rows1, acc.at[dst_v.at[j0 + 1]], add=True)
                if deg_now:
                    @pl.when(c == 0)
                    def _():
                        pltpu.sync_copy(ones_v, dacc.at[dst_v.at[j0 + 1]],
                                        add=True)
                return carry

            lax.fori_loop(0, NPAIR, pair, 0)
            plsc.subcore_barrier()

            pltpu.sync_copy(acc.at[pl.ds(base, RPT)],
                            agg_o.at[q].at[pl.ds(base, RPT)])
            if deg_now:
                @pl.when((c == 0) & (s == 0))
                def _():
                    pltpu.sync_copy(dacc, deg_o)

    return functools.partial(
        pl.kernel, mesh=mesh,
        out_type=tuple(out_type) if len(out_type) > 1 else out_type[0],
        compiler_params=pltpu.CompilerParams(use_tc_tiling_on_sc=False),
        scratch_types=scratch)(body)


def _fold(W, Ad, Bd):
    K = W.shape[1]
    Wb = W.reshape(S, O, K)
    Wm = jnp.einsum('po,spk->sok', Ad, Wb).reshape(S * O, K)
    Wu = jnp.einsum('p,spk->sk', Bd, Wb)
    return Wm, Wu


def kernel(x, h, m, edge_index, W_uxh, b_uxh, Wg_self, Wg_neigh, bg,
           Wc_self, Wc_neigh, bc):
    Ad, Bd = _hippo_constants()
    Wux, Wuh = W_uxh[:D], W_uxh[D:]

    def gate_cols(Wg, bias):
        Wm, Wu = _fold(Wg[D:], Ad, Bd)
        return (Wg[:D] + Wux @ Wu, Wuh @ Wu, Wm, b_uxh @ Wu + bias)

    def cand_xm_cols(Wc, bias):
        Wm, Wu = _fold(Wc[D + H:], Ad, Bd)
        return (Wc[:D] + Wux @ Wu, Wuh @ Wu, Wm, b_uxh @ Wu + bias)

    cols = [gate_cols(Wg_neigh, jnp.zeros((2 * H,), jnp.float32)),
            cand_xm_cols(Wc_neigh, jnp.zeros((H,), jnp.float32)),
            gate_cols(Wg_self, bg),
            cand_xm_cols(Wc_self, bc)]
    Px = jnp.concatenate([cc[0] for cc in cols], axis=1)
    Ph = jnp.concatenate([cc[1] for cc in cols], axis=1)
    Pm = jnp.concatenate([cc[2] for cc in cols], axis=1)
    cv = jnp.concatenate([cc[3] for cc in cols])
    W2 = jnp.concatenate([Wc_neigh[D:D + H], Wc_self[D:D + H]], axis=1)

    m2 = m.reshape(N, S * O)
    y1s, sg, scxm = _mm_a(x, h, m2, Px, Ph, Pm, cv)

    src = edge_index[0].reshape(NS, EPT)
    dst = edge_index[1].reshape(NS, EPT)
    srcT = jnp.pad(src, ((0, 0), (0, EPT_PAD - EPT))).reshape(NS, NCH, CHUNK)
    dstT = jnp.pad(dst, ((0, 0), (0, EPT_PAD - EPT)),
                   constant_values=N).reshape(NS, NCH, CHUNK)

    zeros2 = jnp.zeros((NP, 96), jnp.float32)
    zeros1 = jnp.zeros((NP,), jnp.float32)
    ones_c = jnp.ones((CHUNK,), jnp.float32)
    agg1, deg = _make_sc_segsum(96, 2, True)(y1s, srcT, dstT, zeros2, zeros1,
                                             ones_c)
    degc = deg[:N, None]

    y2s, prec, z = _mm_b(sg, agg1, degc, h, scxm, W2)

    zeros2b = jnp.zeros((NP, 64), jnp.float32)
    agg2 = _make_sc_segsum(64, 1, False)(y2s, srcT, dstT, zeros2b)

    return _mm_c(z, prec, agg2, degc, h)
